# Initial kernel scaffold; baseline (speedup 1.0000x reference)
import jax
import jax.numpy as jnp
from jax.experimental import pallas as pl


def kernel(x, edge_index, batch, params):
    raise NotImplementedError("write your pallas kernel here")


# speedup vs baseline: 4.1043x; 4.1043x over previous
import functools

import jax
import jax.numpy as jnp
from jax import lax
from jax.experimental import pallas as pl
from jax.experimental.pallas import tpu as pltpu
from jax.experimental.pallas import tpu_sc as plsc

N_NODES = 10000
N_EDGES = 320000
D = 128
N_GRAPHS = 64
D_MODEL = 3 * D

NC = 2
NS = 16
NW = NC * NS
EDGES_PER_TILE = N_EDGES // NW
CH = 80
NCH = EDGES_PER_TILE // CH
WB_TILES = 10
WB_ROWS = N_NODES // WB_TILES
ZR = 200

_HI = jax.lax.Precision.HIGHEST


def _sc_agg_body(h_hbm, src_hbm, dst_hbm, out_hbm,
                 accum_sh, src_v, dst_v, rows_v, zero_v, sem):
    c = lax.axis_index("c")
    s = lax.axis_index("s")

    @pl.when(s < WB_TILES)
    def _zero():
        def zb(i, carry):
            for j in range(D // 16):
                zero_v[i, pl.ds(j * 16, 16)] = jnp.zeros((16,), jnp.float32)
            return carry
        lax.fori_loop(0, ZR, zb, 0)
        rbase = s * WB_ROWS
        for j in range(WB_ROWS // ZR):
            pltpu.sync_copy(zero_v, accum_sh.at[pl.ds(rbase + j * ZR, ZR)])
    plsc.subcore_barrier()

    ebase = (c * NS + s) * EDGES_PER_TILE

    def body(k, carry):
        off = pl.multiple_of(ebase + k * CH, 8)
        pltpu.sync_copy(src_hbm.at[pl.ds(off, CH)], src_v)
        pltpu.sync_copy(dst_hbm.at[pl.ds(off, CH)], dst_v)
        pltpu.async_copy(h_hbm.at[src_v], rows_v, sem).wait()
        pltpu.sync_copy(rows_v, accum_sh.at[dst_v], add=True)
        return carry
    lax.fori_loop(0, NCH, body, 0)

    plsc.subcore_barrier()

    @pl.when(s < WB_TILES)
    def _writeback():
        rbase = s * WB_ROWS
        pltpu.sync_copy(accum_sh.at[pl.ds(rbase, WB_ROWS)],
                        out_hbm.at[c, pl.ds(rbase, WB_ROWS)])


@functools.cache
def _make_edge_agg():
    return functools.partial(
        pl.kernel,
        out_type=jax.ShapeDtypeStruct((NC, N_NODES, D), jnp.float32),
        mesh=plsc.VectorSubcoreMesh(core_axis_name="c", subcore_axis_name="s",
                                    num_cores=NC, num_subcores=NS),
        scratch_types=[
            pltpu.VMEM_SHARED((N_NODES, D), jnp.float32),
            pltpu.VMEM((CH,), jnp.int32),
            pltpu.VMEM((CH,), jnp.int32),
            pltpu.VMEM((CH, D), jnp.float32),
            pltpu.VMEM((ZR, D), jnp.float32),
            pltpu.SemaphoreType.DMA,
        ],
    )(_sc_agg_body)


def _edge_agg(h, src, dst):
    return _make_edge_agg()(h, src, dst)


BLK = 1000
NBLK = N_NODES // BLK


def _mlp_block(h_ref, p_ref, bt_ref, w1_ref, b1_ref, w2_ref, b2_ref,
               hout_ref, pool_ref):
    i = pl.program_id(0)
    x = h_ref[...] + p_ref[0] + p_ref[1]
    y = lax.dot_general(x, w1_ref[...], (((1,), (0,)), ((), ())),
                        precision=_HI, preferred_element_type=jnp.float32)
    y = jnp.maximum(y + b1_ref[...], 0.0)
    z = lax.dot_general(y, w2_ref[...], (((1,), (0,)), ((), ())),
                        precision=_HI, preferred_element_type=jnp.float32)
    z = jnp.maximum(z + b2_ref[...], 0.0)
    hout_ref[...] = z
    oh = (bt_ref[...] == lax.broadcasted_iota(jnp.int32, (BLK, N_GRAPHS), 1))
    ohf = oh.astype(jnp.float32)
    pp = lax.dot_general(ohf, z, (((0,), (0,)), ((), ())),
                         precision=_HI, preferred_element_type=jnp.float32)

    @pl.when(i == 0)
    def _():
        pool_ref[...] = pp

    @pl.when(i > 0)
    def _():
        pool_ref[...] = pool_ref[...] + pp


_mlp_call = pl.pallas_call(
    _mlp_block,
    grid=(NBLK,),
    in_specs=[
        pl.BlockSpec((BLK, D), lambda i: (i, 0)),
        pl.BlockSpec((NC, BLK, D), lambda i: (0, i, 0)),
        pl.BlockSpec((BLK, 1), lambda i: (i, 0)),
        pl.BlockSpec((D, D), lambda i: (0, 0)),
        pl.BlockSpec((1, D), lambda i: (0, 0)),
        pl.BlockSpec((D, D), lambda i: (0, 0)),
        pl.BlockSpec((1, D), lambda i: (0, 0)),
    ],
    out_specs=[
        pl.BlockSpec((BLK, D), lambda i: (i, 0)),
        pl.BlockSpec((N_GRAPHS, D), lambda i: (0, 0)),
    ],
    out_shape=[
        jax.ShapeDtypeStruct((N_NODES, D), jnp.float32),
        jax.ShapeDtypeStruct((N_GRAPHS, D), jnp.float32),
    ],
)


def _ln_rows(x, g, b):
    m = jnp.mean(x, axis=1, keepdims=True)
    d = x - m
    v = jnp.mean(d * d, axis=1, keepdims=True)
    return g * (d * lax.rsqrt(v + 1e-5)) + b


def _head_block(hcat_ref, wv_ref, bv_ref, wo_ref, bo_ref,
                g1_ref, be1_ref, wf1_ref, bf1_ref, wf2_ref, bf2_ref,
                g2_ref, be2_ref, wl1_ref, bl1_ref, wl2_ref, bl2_ref,
                out_ref):
    hcat = hcat_ref[...]
    v = lax.dot_general(hcat, wv_ref[...], (((1,), (0,)), ((), ())),
                        precision=_HI, preferred_element_type=jnp.float32)
    v = v + bv_ref[...]
    a = lax.dot_general(v, wo_ref[...], (((1,), (0,)), ((), ())),
                        precision=_HI, preferred_element_type=jnp.float32)
    a = a + bo_ref[...]
    h = _ln_rows(hcat + a, g1_ref[...], be1_ref[...])
    f = lax.dot_general(h, wf1_ref[...], (((1,), (0,)), ((), ())),
                        precision=_HI, preferred_element_type=jnp.float32)
    f = jnp.maximum(f + bf1_ref[...], 0.0)
    f = lax.dot_general(f, wf2_ref[...], (((1,), (0,)), ((), ())),
                        precision=_HI, preferred_element_type=jnp.float32)
    f = f + bf2_ref[...]
    h = _ln_rows(h + f, g2_ref[...], be2_ref[...])
    l = lax.dot_general(h, wl1_ref[...], (((1,), (0,)), ((), ())),
                        precision=_HI, preferred_element_type=jnp.float32)
    l = jnp.maximum(l + bl1_ref[...], 0.0)
    o = lax.dot_general(l, wl2_ref[...], (((1,), (0,)), ((), ())),
                        precision=_HI, preferred_element_type=jnp.float32)
    out_ref[...] = o + bl2_ref[...]


_head_call = pl.pallas_call(
    _head_block,
    out_shape=jax.ShapeDtypeStruct((N_GRAPHS, D), jnp.float32),
)


def _fold_bn(Wa, ba, g, be):
    scale = g / jnp.sqrt(1.0 + 1e-5)
    WT = (Wa * scale[:, None]).T
    b = ba * scale + be
    return WT, b.reshape(1, D)


def kernel(x, edge_index, batch, params):
    P = params
    src = jnp.asarray(edge_index[0], jnp.int32)
    dst = jnp.asarray(edge_index[1], jnp.int32)
    bt2d = jnp.asarray(batch, jnp.int32).reshape(N_NODES, 1)

    W1aT, b1a = _fold_bn(P['W1a'], P['b1a'], P['g1'], P['be1'])
    W1bT, b1b = P['W1b'].T, P['b1b'].reshape(1, D)
    W2aT, b2a = _fold_bn(P['W2a'], P['b2a'], P['g2'], P['be2'])
    W2bT, b2b = P['W2b'].T, P['b2b'].reshape(1, D)

    agg = _edge_agg(x, src, dst)
    h1, pool1 = _mlp_call(x, agg, bt2d, W1aT, b1a, W1bT, b1b)
    agg = _edge_agg(h1, src, dst)
    h2, pool2 = _mlp_call(h1, agg, bt2d, W2aT, b2a, W2bT, b2b)
    agg = _edge_agg(h2, src, dst)
    _, pool3 = _mlp_call(h2, agg, bt2d, W2aT, b2a, W2bT, b2b)

    hcat = jnp.concatenate([pool1, pool2, pool3], axis=1)

    WvT = P['Win'][2 * D_MODEL:].T
    bv = P['bin'][2 * D_MODEL:].reshape(1, D_MODEL)
    WoT = P['Wout'].T
    bo = P['bout'].reshape(1, D_MODEL)
    Wf1T = P['Wff1'].T
    bf1 = P['bff1'].reshape(1, -1)
    Wf2T = P['Wff2'].T
    bf2 = P['bff2'].reshape(1, D_MODEL)
    Wl1T = P['Wl1'].T
    bl1 = P['bl1'].reshape(1, D_MODEL)
    Wl2T = jnp.zeros((D_MODEL, D), jnp.float32).at[:, 0].set(P['Wl2'][0])
    bl2 = jnp.zeros((1, D), jnp.float32).at[0, 0].set(P['bl2'][0])

    out = _head_call(hcat, WvT, bv, WoT, bo,
                     P['ln1g'].reshape(1, -1), P['ln1b'].reshape(1, -1),
                     Wf1T, bf1, Wf2T, bf2,
                     P['ln2g'].reshape(1, -1), P['ln2b'].reshape(1, -1),
                     Wl1T, bl1, Wl2T, bl2)
    return out[:, :1]


# speedup vs baseline: 8.5135x; 2.0743x over previous
import functools

import jax
import jax.numpy as jnp
from jax import lax
from jax.experimental import pallas as pl
from jax.experimental.pallas import tpu as pltpu
from jax.experimental.pallas import tpu_sc as plsc

N_NODES = 10000
N_EDGES = 320000
D = 128
N_GRAPHS = 64
D_MODEL = 3 * D

NC = 2
NS = 16
NW = NC * NS
CH = 128
CPT = 78
TAIL_BASE = NW * CPT * CH
N_TAIL = (N_EDGES - TAIL_BASE) // CH
WB_TILES = 10
WB_ROWS = N_NODES // WB_TILES
ZR = 40

_HI = jax.lax.Precision.HIGHEST


def _sc_agg_body(h_hbm, src_hbm, dst_hbm, out_hbm,
                 accum_sh, src_v0, src_v1, dst_v0, dst_v1,
                 rows_v0, rows_v1, zero_v,
                 ssem0, ssem1, dsem0, dsem1, gsem0, gsem1):
    c = lax.axis_index("c")
    s = lax.axis_index("s")
    w = c * NS + s
    ebase = w * (CPT * CH)

    src_v = (src_v0, src_v1)
    dst_v = (dst_v0, dst_v1)
    rows_v = (rows_v0, rows_v1)
    ssem = (ssem0, ssem1)
    dsem = (dsem0, dsem1)
    gsem = (gsem0, gsem1)

    def _eoff(k):
        return pl.multiple_of(ebase + k * CH, CH)

    def _fire_idx(k, j):
        pltpu.async_copy(src_hbm.at[pl.ds(_eoff(k), CH)], src_v[j], ssem[j])
        pltpu.async_copy(dst_hbm.at[pl.ds(_eoff(k), CH)], dst_v[j], dsem[j])

    def _wait(buf, sem_):
        pltpu.make_async_copy(src_hbm.at[pl.ds(0, CH)], buf, sem_).wait()

    def _wait_rows(j):
        pltpu.make_async_copy(h_hbm.at[pl.ds(0, CH)], rows_v[j], gsem[j]).wait()

    _fire_idx(0, 0)
    _fire_idx(1, 1)

    @pl.when(s < WB_TILES)
    def _zero():
        def zb(i, carry):
            for j in range(D // 16):
                zero_v[i, pl.ds(j * 16, 16)] = jnp.zeros((16,), jnp.float32)
            return carry
        lax.fori_loop(0, ZR, zb, 0)
        rbase = s * WB_ROWS
        for j in range(WB_ROWS // ZR):
            pltpu.sync_copy(zero_v, accum_sh.at[pl.ds(rbase + j * ZR, ZR)])

    _wait(src_v[0], ssem[0])
    pltpu.async_copy(h_hbm.at[src_v0], rows_v0, gsem[0])
    plsc.subcore_barrier()

    def _step(k, j, fire_gather, fire_idx):
        j1 = 1 - j
        if fire_gather:
            _wait(src_v[j1], ssem[j1])
            pltpu.async_copy(h_hbm.at[src_v[j1]], rows_v[j1], gsem[j1])
        _wait_rows(j)
        _wait(dst_v[j], dsem[j])
        pltpu.sync_copy(rows_v[j], accum_sh.at[dst_v[j]], add=True)
        if fire_idx:
            _fire_idx(k + 2, j)

    def body(i, carry):
        _step(2 * i, 0, True, True)
        _step(2 * i + 1, 1, True, True)
        return carry
    lax.fori_loop(0, CPT // 2 - 1, body, 0)
    _step(CPT - 2, 0, True, False)
    _step(CPT - 1, 1, False, False)

    @pl.when(jnp.logical_and(c == 0, s < N_TAIL))
    def _tail():
        toff = pl.multiple_of(TAIL_BASE + s * CH, CH)
        pltpu.sync_copy(src_hbm.at[pl.ds(toff, CH)], src_v0)
        pltpu.sync_copy(dst_hbm.at[pl.ds(toff, CH)], dst_v0)
        pltpu.async_copy(h_hbm.at[src_v0], rows_v0, gsem[0]).wait()
        pltpu.sync_copy(rows_v0, accum_sh.at[dst_v0], add=True)

    plsc.subcore_barrier()

    @pl.when(s < WB_TILES)
    def _writeback():
        rbase = s * WB_ROWS
        pltpu.sync_copy(accum_sh.at[pl.ds(rbase, WB_ROWS)],
                        out_hbm.at[c, pl.ds(rbase, WB_ROWS)])


@functools.cache
def _make_edge_agg():
    return functools.partial(
        pl.kernel,
        out_type=jax.ShapeDtypeStruct((NC, N_NODES, D), jnp.float32),
        mesh=plsc.VectorSubcoreMesh(core_axis_name="c", subcore_axis_name="s",
                                    num_cores=NC, num_subcores=NS),
        scratch_types=[
            pltpu.VMEM_SHARED((N_NODES, D), jnp.float32),
            pltpu.VMEM((CH,), jnp.int32),
            pltpu.VMEM((CH,), jnp.int32),
            pltpu.VMEM((CH,), jnp.int32),
            pltpu.VMEM((CH,), jnp.int32),
            pltpu.VMEM((CH, D), jnp.float32),
            pltpu.VMEM((CH, D), jnp.float32),
            pltpu.VMEM((ZR, D), jnp.float32),
            pltpu.SemaphoreType.DMA,
            pltpu.SemaphoreType.DMA,
            pltpu.SemaphoreType.DMA,
            pltpu.SemaphoreType.DMA,
            pltpu.SemaphoreType.DMA,
            pltpu.SemaphoreType.DMA,
        ],
    )(_sc_agg_body)


def _edge_agg(h, src, dst):
    return _make_edge_agg()(h, src, dst)


BLK = 1000
NBLK = N_NODES // BLK


def _oh_block(bt_ref, oh_ref):
    oh = (bt_ref[...] == lax.broadcasted_iota(jnp.int32, (BLK, N_GRAPHS), 1))
    oh_ref[...] = oh.astype(jnp.float32)


_oh_call = pl.pallas_call(
    _oh_block,
    grid=(NBLK,),
    in_specs=[pl.BlockSpec((BLK, 1), lambda i: (i, 0))],
    out_specs=pl.BlockSpec((BLK, N_GRAPHS), lambda i: (i, 0)),
    out_shape=jax.ShapeDtypeStruct((N_NODES, N_GRAPHS), jnp.float32),
)


def _mlp_block(h_ref, p_ref, oh_ref, w1_ref, b1_ref, w2_ref, b2_ref,
               hout_ref, pool_ref):
    i = pl.program_id(0)
    x = h_ref[...] + p_ref[0] + p_ref[1]
    y = lax.dot_general(x, w1_ref[...], (((1,), (0,)), ((), ())),
                        precision=_HI, preferred_element_type=jnp.float32)
    y = jnp.maximum(y + b1_ref[...], 0.0)
    z = lax.dot_general(y, w2_ref[...], (((1,), (0,)), ((), ())),
                        precision=_HI, preferred_element_type=jnp.float32)
    z = jnp.maximum(z + b2_ref[...], 0.0)
    hout_ref[...] = z
    pp = lax.dot_general(oh_ref[...], z, (((0,), (0,)), ((), ())),
                         precision=_HI, preferred_element_type=jnp.float32)

    @pl.when(i == 0)
    def _():
        pool_ref[...] = pp

    @pl.when(i > 0)
    def _():
        pool_ref[...] = pool_ref[...] + pp


_mlp_call = pl.pallas_call(
    _mlp_block,
    grid=(NBLK,),
    in_specs=[
        pl.BlockSpec((BLK, D), lambda i: (i, 0)),
        pl.BlockSpec((NC, BLK, D), lambda i: (0, i, 0)),
        pl.BlockSpec((BLK, N_GRAPHS), lambda i: (i, 0)),
        pl.BlockSpec((D, D), lambda i: (0, 0)),
        pl.BlockSpec((1, D), lambda i: (0, 0)),
        pl.BlockSpec((D, D), lambda i: (0, 0)),
        pl.BlockSpec((1, D), lambda i: (0, 0)),
    ],
    out_specs=[
        pl.BlockSpec((BLK, D), lambda i: (i, 0)),
        pl.BlockSpec((N_GRAPHS, D), lambda i: (0, 0)),
    ],
    out_shape=[
        jax.ShapeDtypeStruct((N_NODES, D), jnp.float32),
        jax.ShapeDtypeStruct((N_GRAPHS, D), jnp.float32),
    ],
)


def _ln_rows(x, g, b):
    m = jnp.mean(x, axis=1, keepdims=True)
    d = x - m
    v = jnp.mean(d * d, axis=1, keepdims=True)
    return g * (d * lax.rsqrt(v + 1e-5)) + b


def _head_block(hcat_ref, wv_ref, bv_ref, wo_ref, bo_ref,
                g1_ref, be1_ref, wf1_ref, bf1_ref, wf2_ref, bf2_ref,
                g2_ref, be2_ref, wl1_ref, bl1_ref, wl2_ref, bl2_ref,
                out_ref):
    hcat = hcat_ref[...]
    v = lax.dot_general(hcat, wv_ref[...], (((1,), (0,)), ((), ())),
                        precision=_HI, preferred_element_type=jnp.float32)
    v = v + bv_ref[...]
    a = lax.dot_general(v, wo_ref[...], (((1,), (0,)), ((), ())),
                        precision=_HI, preferred_element_type=jnp.float32)
    a = a + bo_ref[...]
    h = _ln_rows(hcat + a, g1_ref[...], be1_ref[...])
    f = lax.dot_general(h, wf1_ref[...], (((1,), (0,)), ((), ())),
                        precision=_HI, preferred_element_type=jnp.float32)
    f = jnp.maximum(f + bf1_ref[...], 0.0)
    f = lax.dot_general(f, wf2_ref[...], (((1,), (0,)), ((), ())),
                        precision=_HI, preferred_element_type=jnp.float32)
    f = f + bf2_ref[...]
    h = _ln_rows(h + f, g2_ref[...], be2_ref[...])
    l = lax.dot_general(h, wl1_ref[...], (((1,), (0,)), ((), ())),
                        precision=_HI, preferred_element_type=jnp.float32)
    l = jnp.maximum(l + bl1_ref[...], 0.0)
    o = lax.dot_general(l, wl2_ref[...], (((1,), (0,)), ((), ())),
                        precision=_HI, preferred_element_type=jnp.float32)
    out_ref[...] = o + bl2_ref[...]


_head_call = pl.pallas_call(
    _head_block,
    out_shape=jax.ShapeDtypeStruct((N_GRAPHS, D), jnp.float32),
)


def _fold_bn(Wa, ba, g, be):
    scale = g / jnp.sqrt(1.0 + 1e-5)
    WT = (Wa * scale[:, None]).T
    b = ba * scale + be
    return WT, b.reshape(1, D)


def kernel(x, edge_index, batch, params):
    P = params
    src = jnp.asarray(edge_index[0], jnp.int32)
    dst = jnp.asarray(edge_index[1], jnp.int32)
    bt2d = jnp.asarray(batch, jnp.int32).reshape(N_NODES, 1)

    W1aT, b1a = _fold_bn(P['W1a'], P['b1a'], P['g1'], P['be1'])
    W1bT, b1b = P['W1b'].T, P['b1b'].reshape(1, D)
    W2aT, b2a = _fold_bn(P['W2a'], P['b2a'], P['g2'], P['be2'])
    W2bT, b2b = P['W2b'].T, P['b2b'].reshape(1, D)

    oh = _oh_call(bt2d)
    agg = _edge_agg(x, src, dst)
    h1, pool1 = _mlp_call(x, agg, oh, W1aT, b1a, W1bT, b1b)
    agg = _edge_agg(h1, src, dst)
    h2, pool2 = _mlp_call(h1, agg, oh, W2aT, b2a, W2bT, b2b)
    agg = _edge_agg(h2, src, dst)
    _, pool3 = _mlp_call(h2, agg, oh, W2aT, b2a, W2bT, b2b)

    hcat = jnp.concatenate([pool1, pool2, pool3], axis=1)

    WvT = P['Win'][2 * D_MODEL:].T
    bv = P['bin'][2 * D_MODEL:].reshape(1, D_MODEL)
    WoT = P['Wout'].T
    bo = P['bout'].reshape(1, D_MODEL)
    Wf1T = P['Wff1'].T
    bf1 = P['bff1'].reshape(1, -1)
    Wf2T = P['Wff2'].T
    bf2 = P['bff2'].reshape(1, D_MODEL)
    Wl1T = P['Wl1'].T
    bl1 = P['bl1'].reshape(1, D_MODEL)
    Wl2T = jnp.zeros((D_MODEL, D), jnp.float32).at[:, 0].set(P['Wl2'][0])
    bl2 = jnp.zeros((1, D), jnp.float32).at[0, 0].set(P['bl2'][0])

    out = _head_call(hcat, WvT, bv, WoT, bo,
                     P['ln1g'].reshape(1, -1), P['ln1b'].reshape(1, -1),
                     Wf1T, bf1, Wf2T, bf2,
                     P['ln2g'].reshape(1, -1), P['ln2b'].reshape(1, -1),
                     Wl1T, bl1, Wl2T, bl2)
    return out[:, :1]


# speedup vs baseline: 9.2804x; 1.0901x over previous
import functools

import jax
import jax.numpy as jnp
from jax import lax
from jax.experimental import pallas as pl
from jax.experimental.pallas import tpu as pltpu
from jax.experimental.pallas import tpu_sc as plsc

N_NODES = 10000
N_EDGES = 320000
D = 128
N_GRAPHS = 64
D_MODEL = 3 * D

NC = 2
NS = 16
NW = NC * NS
CH = 128
CPT = 78
TAIL_BASE = NW * CPT * CH
N_TAIL = (N_EDGES - TAIL_BASE) // CH
WB_TILES = 10
WB_ROWS = N_NODES // WB_TILES
ZR = 40

_HI = jax.lax.Precision.HIGHEST


def _sc_agg_body(h_hbm, src_hbm, dst_hbm, out_hbm,
                 accum_sh, src_v0, src_v1, dst_v0, dst_v1,
                 rows_v0, rows_v1, zero_v,
                 ssem0, ssem1, dsem0, dsem1, gsem0, gsem1):
    c = lax.axis_index("c")
    s = lax.axis_index("s")
    w = c * NS + s
    ebase = w * (CPT * CH)

    src_v = (src_v0, src_v1)
    dst_v = (dst_v0, dst_v1)
    rows_v = (rows_v0, rows_v1)
    ssem = (ssem0, ssem1)
    dsem = (dsem0, dsem1)
    gsem = (gsem0, gsem1)

    def _eoff(k):
        return pl.multiple_of(ebase + k * CH, CH)

    def _fire_idx(k, j):
        pltpu.async_copy(src_hbm.at[pl.ds(_eoff(k), CH)], src_v[j], ssem[j])
        pltpu.async_copy(dst_hbm.at[pl.ds(_eoff(k), CH)], dst_v[j], dsem[j])

    def _wait(buf, sem_):
        pltpu.make_async_copy(src_hbm.at[pl.ds(0, CH)], buf, sem_).wait()

    def _wait_rows(j):
        pltpu.make_async_copy(h_hbm.at[pl.ds(0, CH)], rows_v[j], gsem[j]).wait()

    _fire_idx(0, 0)
    _fire_idx(1, 1)

    @pl.when(s < WB_TILES)
    def _zero():
        def zb(i, carry):
            for j in range(D // 16):
                zero_v[i, pl.ds(j * 16, 16)] = jnp.zeros((16,), jnp.float32)
            return carry
        lax.fori_loop(0, ZR, zb, 0)
        rbase = s * WB_ROWS
        for j in range(WB_ROWS // ZR):
            pltpu.sync_copy(zero_v, accum_sh.at[pl.ds(rbase + j * ZR, ZR)])

    _wait(src_v[0], ssem[0])
    pltpu.async_copy(h_hbm.at[src_v0], rows_v0, gsem[0])
    plsc.subcore_barrier()

    def _step(k, j, fire_gather, fire_idx):
        j1 = 1 - j
        if fire_gather:
            _wait(src_v[j1], ssem[j1])
            pltpu.async_copy(h_hbm.at[src_v[j1]], rows_v[j1], gsem[j1])
        _wait_rows(j)
        _wait(dst_v[j], dsem[j])
        pltpu.sync_copy(rows_v[j], accum_sh.at[dst_v[j]], add=True)
        if fire_idx:
            _fire_idx(k + 2, j)

    def body(i, carry):
        _step(2 * i, 0, True, True)
        _step(2 * i + 1, 1, True, True)
        return carry
    lax.fori_loop(0, CPT // 2 - 1, body, 0)
    _step(CPT - 2, 0, True, False)
    _step(CPT - 1, 1, False, False)

    @pl.when(jnp.logical_and(c == 0, s < N_TAIL))
    def _tail():
        toff = pl.multiple_of(TAIL_BASE + s * CH, CH)
        pltpu.sync_copy(src_hbm.at[pl.ds(toff, CH)], src_v0)
        pltpu.sync_copy(dst_hbm.at[pl.ds(toff, CH)], dst_v0)
        pltpu.async_copy(h_hbm.at[src_v0], rows_v0, gsem[0]).wait()
        pltpu.sync_copy(rows_v0, accum_sh.at[dst_v0], add=True)

    plsc.subcore_barrier()

    @pl.when(s < WB_TILES)
    def _writeback():
        rbase = s * WB_ROWS
        pltpu.sync_copy(accum_sh.at[pl.ds(rbase, WB_ROWS)],
                        out_hbm.at[c, pl.ds(rbase, WB_ROWS)])


@functools.cache
def _make_edge_agg():
    return functools.partial(
        pl.kernel,
        out_type=jax.ShapeDtypeStruct((NC, N_NODES, D), jnp.float32),
        mesh=plsc.VectorSubcoreMesh(core_axis_name="c", subcore_axis_name="s",
                                    num_cores=NC, num_subcores=NS),
        scratch_types=[
            pltpu.VMEM_SHARED((N_NODES, D), jnp.float32),
            pltpu.VMEM((CH,), jnp.int32),
            pltpu.VMEM((CH,), jnp.int32),
            pltpu.VMEM((CH,), jnp.int32),
            pltpu.VMEM((CH,), jnp.int32),
            pltpu.VMEM((CH, D), jnp.float32),
            pltpu.VMEM((CH, D), jnp.float32),
            pltpu.VMEM((ZR, D), jnp.float32),
            pltpu.SemaphoreType.DMA,
            pltpu.SemaphoreType.DMA,
            pltpu.SemaphoreType.DMA,
            pltpu.SemaphoreType.DMA,
            pltpu.SemaphoreType.DMA,
            pltpu.SemaphoreType.DMA,
        ],
    )(_sc_agg_body)


def _edge_agg(h, src, dst):
    return _make_edge_agg()(h, src, dst)


BLK = 2000
NBLK = N_NODES // BLK


def _oh_block(bt_ref, oh_ref):
    oh = (bt_ref[...] == lax.broadcasted_iota(jnp.int32, (BLK, N_GRAPHS), 1))
    oh_ref[...] = oh.astype(jnp.float32)


_oh_call = pl.pallas_call(
    _oh_block,
    grid=(NBLK,),
    in_specs=[pl.BlockSpec((BLK, 1), lambda i: (i, 0))],
    out_specs=pl.BlockSpec((BLK, N_GRAPHS), lambda i: (i, 0)),
    out_shape=jax.ShapeDtypeStruct((N_NODES, N_GRAPHS), jnp.float32),
)


def _mlp_block(h_ref, p_ref, oh_ref, w1_ref, b1_ref, w2_ref, b2_ref,
               hout_ref, pool_ref):
    i = pl.program_id(0)
    x = h_ref[...] + p_ref[0] + p_ref[1]
    y = lax.dot_general(x, w1_ref[...], (((1,), (0,)), ((), ())),
                        precision=_HI, preferred_element_type=jnp.float32)
    y = jnp.maximum(y + b1_ref[...], 0.0)
    z = lax.dot_general(y, w2_ref[...], (((1,), (0,)), ((), ())),
                        precision=_HI, preferred_element_type=jnp.float32)
    z = jnp.maximum(z + b2_ref[...], 0.0)
    hout_ref[...] = z
    pp = lax.dot_general(oh_ref[...], z, (((0,), (0,)), ((), ())),
                         precision=_HI, preferred_element_type=jnp.float32)

    @pl.when(i == 0)
    def _():
        pool_ref[...] = pp

    @pl.when(i > 0)
    def _():
        pool_ref[...] = pool_ref[...] + pp


_mlp_call = pl.pallas_call(
    _mlp_block,
    grid=(NBLK,),
    in_specs=[
        pl.BlockSpec((BLK, D), lambda i: (i, 0)),
        pl.BlockSpec((NC, BLK, D), lambda i: (0, i, 0)),
        pl.BlockSpec((BLK, N_GRAPHS), lambda i: (i, 0)),
        pl.BlockSpec((D, D), lambda i: (0, 0)),
        pl.BlockSpec((1, D), lambda i: (0, 0)),
        pl.BlockSpec((D, D), lambda i: (0, 0)),
        pl.BlockSpec((1, D), lambda i: (0, 0)),
    ],
    out_specs=[
        pl.BlockSpec((BLK, D), lambda i: (i, 0)),
        pl.BlockSpec((N_GRAPHS, D), lambda i: (0, 0)),
    ],
    out_shape=[
        jax.ShapeDtypeStruct((N_NODES, D), jnp.float32),
        jax.ShapeDtypeStruct((N_GRAPHS, D), jnp.float32),
    ],
)


def _ln_rows(x, g, b):
    m = jnp.mean(x, axis=1, keepdims=True)
    d = x - m
    v = jnp.mean(d * d, axis=1, keepdims=True)
    return g * (d * lax.rsqrt(v + 1e-5)) + b


def _head_block(hcat_ref, wv_ref, bv_ref, wo_ref, bo_ref,
                g1_ref, be1_ref, wf1_ref, bf1_ref, wf2_ref, bf2_ref,
                g2_ref, be2_ref, wl1_ref, bl1_ref, wl2_ref, bl2_ref,
                out_ref):
    hcat = hcat_ref[...]
    v = lax.dot_general(hcat, wv_ref[...], (((1,), (0,)), ((), ())),
                        precision=_HI, preferred_element_type=jnp.float32)
    v = v + bv_ref[...]
    a = lax.dot_general(v, wo_ref[...], (((1,), (0,)), ((), ())),
                        precision=_HI, preferred_element_type=jnp.float32)
    a = a + bo_ref[...]
    h = _ln_rows(hcat + a, g1_ref[...], be1_ref[...])
    f = lax.dot_general(h, wf1_ref[...], (((1,), (0,)), ((), ())),
                        precision=_HI, preferred_element_type=jnp.float32)
    f = jnp.maximum(f + bf1_ref[...], 0.0)
    f = lax.dot_general(f, wf2_ref[...], (((1,), (0,)), ((), ())),
                        precision=_HI, preferred_element_type=jnp.float32)
    f = f + bf2_ref[...]
    h = _ln_rows(h + f, g2_ref[...], be2_ref[...])
    l = lax.dot_general(h, wl1_ref[...], (((1,), (0,)), ((), ())),
                        precision=_HI, preferred_element_type=jnp.float32)
    l = jnp.maximum(l + bl1_ref[...], 0.0)
    o = lax.dot_general(l, wl2_ref[...], (((1,), (0,)), ((), ())),
                        precision=_HI, preferred_element_type=jnp.float32)
    out_ref[...] = o + bl2_ref[...]


_head_call = pl.pallas_call(
    _head_block,
    out_shape=jax.ShapeDtypeStruct((N_GRAPHS, D), jnp.float32),
)


def _fold_bn(Wa, ba, g, be):
    scale = g / jnp.sqrt(1.0 + 1e-5)
    WT = (Wa * scale[:, None]).T
    b = ba * scale + be
    return WT, b.reshape(1, D)


def kernel(x, edge_index, batch, params):
    P = params
    src = jnp.asarray(edge_index[0], jnp.int32)
    dst = jnp.asarray(edge_index[1], jnp.int32)
    bt2d = jnp.asarray(batch, jnp.int32).reshape(N_NODES, 1)

    W1aT, b1a = _fold_bn(P['W1a'], P['b1a'], P['g1'], P['be1'])
    W1bT, b1b = P['W1b'].T, P['b1b'].reshape(1, D)
    W2aT, b2a = _fold_bn(P['W2a'], P['b2a'], P['g2'], P['be2'])
    W2bT, b2b = P['W2b'].T, P['b2b'].reshape(1, D)

    oh = _oh_call(bt2d)
    agg = _edge_agg(x, src, dst)
    h1, pool1 = _mlp_call(x, agg, oh, W1aT, b1a, W1bT, b1b)
    agg = _edge_agg(h1, src, dst)
    h2, pool2 = _mlp_call(h1, agg, oh, W2aT, b2a, W2bT, b2b)
    agg = _edge_agg(h2, src, dst)
    _, pool3 = _mlp_call(h2, agg, oh, W2aT, b2a, W2bT, b2b)

    hcat = jnp.concatenate([pool1, pool2, pool3], axis=1)

    WvT = P['Win'][2 * D_MODEL:].T
    bv = P['bin'][2 * D_MODEL:].reshape(1, D_MODEL)
    WoT = P['Wout'].T
    bo = P['bout'].reshape(1, D_MODEL)
    Wf1T = P['Wff1'].T
    bf1 = P['bff1'].reshape(1, -1)
    Wf2T = P['Wff2'].T
    bf2 = P['bff2'].reshape(1, D_MODEL)
    Wl1T = P['Wl1'].T
    bl1 = P['bl1'].reshape(1, D_MODEL)
    Wl2T = jnp.zeros((D_MODEL, D), jnp.float32).at[:, 0].set(P['Wl2'][0])
    bl2 = jnp.zeros((1, D), jnp.float32).at[0, 0].set(P['bl2'][0])

    out = _head_call(hcat, WvT, bv, WoT, bo,
                     P['ln1g'].reshape(1, -1), P['ln1b'].reshape(1, -1),
                     Wf1T, bf1, Wf2T, bf2,
                     P['ln2g'].reshape(1, -1), P['ln2b'].reshape(1, -1),
                     Wl1T, bl1, Wl2T, bl2)
    return out[:, :1]


# speedup vs baseline: 10.1017x; 1.0885x over previous
import functools

import jax
import jax.numpy as jnp
from jax import lax
from jax.experimental import pallas as pl
from jax.experimental.pallas import tpu as pltpu
from jax.experimental.pallas import tpu_sc as plsc

N_NODES = 10000
N_EDGES = 320000
D = 128
N_GRAPHS = 64
D_MODEL = 3 * D

NC = 2
NS = 16
NW = NC * NS
CH = 128
CPT = 78
TAIL_BASE = NW * CPT * CH
N_TAIL = (N_EDGES - TAIL_BASE) // CH
WB_TILES = 10
WB_ROWS = N_NODES // WB_TILES
ZR = 40


def _sc_agg_body(h_hbm, src_hbm, dst_hbm, out_hbm,
                 accum_sh, src_v0, src_v1, dst_v0, dst_v1,
                 rows_v0, rows_v1, zero_v,
                 ssem0, ssem1, dsem0, dsem1, gsem0, gsem1):
    c = lax.axis_index("c")
    s = lax.axis_index("s")
    w = c * NS + s
    ebase = w * (CPT * CH)

    src_v = (src_v0, src_v1)
    dst_v = (dst_v0, dst_v1)
    rows_v = (rows_v0, rows_v1)
    ssem = (ssem0, ssem1)
    dsem = (dsem0, dsem1)
    gsem = (gsem0, gsem1)

    def _eoff(k):
        return pl.multiple_of(ebase + k * CH, CH)

    def _fire_idx(k, j):
        pltpu.async_copy(src_hbm.at[pl.ds(_eoff(k), CH)], src_v[j], ssem[j])
        pltpu.async_copy(dst_hbm.at[pl.ds(_eoff(k), CH)], dst_v[j], dsem[j])

    def _wait(buf, sem_):
        pltpu.make_async_copy(src_hbm.at[pl.ds(0, CH)], buf, sem_).wait()

    def _wait_rows(j):
        pltpu.make_async_copy(h_hbm.at[pl.ds(0, CH)], rows_v[j], gsem[j]).wait()

    _fire_idx(0, 0)
    _fire_idx(1, 1)

    @pl.when(s < WB_TILES)
    def _zero():
        def zb(i, carry):
            for j in range(D // 16):
                zero_v[i, pl.ds(j * 16, 16)] = jnp.zeros((16,), jnp.float32)
            return carry
        lax.fori_loop(0, ZR, zb, 0)
        rbase = s * WB_ROWS
        for j in range(WB_ROWS // ZR):
            pltpu.sync_copy(zero_v, accum_sh.at[pl.ds(rbase + j * ZR, ZR)])

    _wait(src_v[0], ssem[0])
    pltpu.async_copy(h_hbm.at[src_v0], rows_v0, gsem[0])
    plsc.subcore_barrier()

    def _step(k, j, fire_gather, fire_idx):
        j1 = 1 - j
        if fire_gather:
            _wait(src_v[j1], ssem[j1])
            pltpu.async_copy(h_hbm.at[src_v[j1]], rows_v[j1], gsem[j1])
        _wait_rows(j)
        _wait(dst_v[j], dsem[j])
        pltpu.sync_copy(rows_v[j], accum_sh.at[dst_v[j]], add=True)
        if fire_idx:
            _fire_idx(k + 2, j)

    def body(i, carry):
        _step(2 * i, 0, True, True)
        _step(2 * i + 1, 1, True, True)
        return carry
    lax.fori_loop(0, CPT // 2 - 1, body, 0)
    _step(CPT - 2, 0, True, False)
    _step(CPT - 1, 1, False, False)

    @pl.when(jnp.logical_and(c == 0, s < N_TAIL))
    def _tail():
        toff = pl.multiple_of(TAIL_BASE + s * CH, CH)
        pltpu.sync_copy(src_hbm.at[pl.ds(toff, CH)], src_v0)
        pltpu.sync_copy(dst_hbm.at[pl.ds(toff, CH)], dst_v0)
        pltpu.async_copy(h_hbm.at[src_v0], rows_v0, gsem[0]).wait()
        pltpu.sync_copy(rows_v0, accum_sh.at[dst_v0], add=True)

    plsc.subcore_barrier()

    @pl.when(s < WB_TILES)
    def _writeback():
        rbase = s * WB_ROWS
        pltpu.sync_copy(accum_sh.at[pl.ds(rbase, WB_ROWS)],
                        out_hbm.at[c, pl.ds(rbase, WB_ROWS)])


@functools.cache
def _make_edge_agg():
    return functools.partial(
        pl.kernel,
        out_type=jax.ShapeDtypeStruct((NC, N_NODES, D), jnp.float32),
        mesh=plsc.VectorSubcoreMesh(core_axis_name="c", subcore_axis_name="s",
                                    num_cores=NC, num_subcores=NS),
        scratch_types=[
            pltpu.VMEM_SHARED((N_NODES, D), jnp.float32),
            pltpu.VMEM((CH,), jnp.int32),
            pltpu.VMEM((CH,), jnp.int32),
            pltpu.VMEM((CH,), jnp.int32),
            pltpu.VMEM((CH,), jnp.int32),
            pltpu.VMEM((CH, D), jnp.float32),
            pltpu.VMEM((CH, D), jnp.float32),
            pltpu.VMEM((ZR, D), jnp.float32),
            pltpu.SemaphoreType.DMA,
            pltpu.SemaphoreType.DMA,
            pltpu.SemaphoreType.DMA,
            pltpu.SemaphoreType.DMA,
            pltpu.SemaphoreType.DMA,
            pltpu.SemaphoreType.DMA,
        ],
    )(_sc_agg_body)


def _edge_agg(h, src, dst):
    return _make_edge_agg()(h, src, dst)


BLK = 2000
NBLK = N_NODES // BLK


def _oh_block(bt_ref, oh_ref):
    oh = (bt_ref[...] == lax.broadcasted_iota(jnp.int32, (BLK, N_GRAPHS), 1))
    oh_ref[...] = oh.astype(jnp.float32)


_oh_call = pl.pallas_call(
    _oh_block,
    grid=(NBLK,),
    in_specs=[pl.BlockSpec((BLK, 1), lambda i: (i, 0))],
    out_specs=pl.BlockSpec((BLK, N_GRAPHS), lambda i: (i, 0)),
    out_shape=jax.ShapeDtypeStruct((N_NODES, N_GRAPHS), jnp.float32),
)


def _mlp_block(h_ref, p_ref, oh_ref, w1_ref, b1_ref, w2_ref, b2_ref,
               hout_ref, pool_ref):
    i = pl.program_id(0)
    x = h_ref[...] + p_ref[0] + p_ref[1]
    y = lax.dot_general(x, w1_ref[...], (((1,), (0,)), ((), ())),
                        preferred_element_type=jnp.float32)
    y = jnp.maximum(y + b1_ref[...], 0.0)
    z = lax.dot_general(y, w2_ref[...], (((1,), (0,)), ((), ())),
                        preferred_element_type=jnp.float32)
    z = jnp.maximum(z + b2_ref[...], 0.0)
    hout_ref[...] = z
    pp = lax.dot_general(oh_ref[...], z, (((0,), (0,)), ((), ())),
                         preferred_element_type=jnp.float32)

    @pl.when(i == 0)
    def _():
        pool_ref[...] = pp

    @pl.when(i > 0)
    def _():
        pool_ref[...] = pool_ref[...] + pp


_mlp_call = pl.pallas_call(
    _mlp_block,
    grid=(NBLK,),
    in_specs=[
        pl.BlockSpec((BLK, D), lambda i: (i, 0)),
        pl.BlockSpec((NC, BLK, D), lambda i: (0, i, 0)),
        pl.BlockSpec((BLK, N_GRAPHS), lambda i: (i, 0)),
        pl.BlockSpec((D, D), lambda i: (0, 0)),
        pl.BlockSpec((1, D), lambda i: (0, 0)),
        pl.BlockSpec((D, D), lambda i: (0, 0)),
        pl.BlockSpec((1, D), lambda i: (0, 0)),
    ],
    out_specs=[
        pl.BlockSpec((BLK, D), lambda i: (i, 0)),
        pl.BlockSpec((N_GRAPHS, D), lambda i: (0, 0)),
    ],
    out_shape=[
        jax.ShapeDtypeStruct((N_NODES, D), jnp.float32),
        jax.ShapeDtypeStruct((N_GRAPHS, D), jnp.float32),
    ],
)


def _ln_rows(x, g, b):
    m = jnp.mean(x, axis=1, keepdims=True)
    d = x - m
    v = jnp.mean(d * d, axis=1, keepdims=True)
    return g * (d * lax.rsqrt(v + 1e-5)) + b


def _head_block(hcat_ref, wv_ref, bv_ref, wo_ref, bo_ref,
                g1_ref, be1_ref, wf1_ref, bf1_ref, wf2_ref, bf2_ref,
                g2_ref, be2_ref, wl1_ref, bl1_ref, wl2_ref, bl2_ref,
                out_ref):
    hcat = hcat_ref[...]
    v = lax.dot_general(hcat, wv_ref[...], (((1,), (0,)), ((), ())),
                        preferred_element_type=jnp.float32)
    v = v + bv_ref[...]
    a = lax.dot_general(v, wo_ref[...], (((1,), (0,)), ((), ())),
                        preferred_element_type=jnp.float32)
    a = a + bo_ref[...]
    h = _ln_rows(hcat + a, g1_ref[...], be1_ref[...])
    f = lax.dot_general(h, wf1_ref[...], (((1,), (0,)), ((), ())),
                        preferred_element_type=jnp.float32)
    f = jnp.maximum(f + bf1_ref[...], 0.0)
    f = lax.dot_general(f, wf2_ref[...], (((1,), (0,)), ((), ())),
                        preferred_element_type=jnp.float32)
    f = f + bf2_ref[...]
    h = _ln_rows(h + f, g2_ref[...], be2_ref[...])
    l = lax.dot_general(h, wl1_ref[...], (((1,), (0,)), ((), ())),
                        preferred_element_type=jnp.float32)
    l = jnp.maximum(l + bl1_ref[...], 0.0)
    o = lax.dot_general(l, wl2_ref[...], (((1,), (0,)), ((), ())),
                        preferred_element_type=jnp.float32)
    out_ref[...] = o + bl2_ref[...]


_head_call = pl.pallas_call(
    _head_block,
    out_shape=jax.ShapeDtypeStruct((N_GRAPHS, D), jnp.float32),
)


def _fold_bn(Wa, ba, g, be):
    scale = g / jnp.sqrt(1.0 + 1e-5)
    WT = (Wa * scale[:, None]).T
    b = ba * scale + be
    return WT, b.reshape(1, D)


def kernel(x, edge_index, batch, params):
    P = params
    src = jnp.asarray(edge_index[0], jnp.int32)
    dst = jnp.asarray(edge_index[1], jnp.int32)
    bt2d = jnp.asarray(batch, jnp.int32).reshape(N_NODES, 1)

    W1aT, b1a = _fold_bn(P['W1a'], P['b1a'], P['g1'], P['be1'])
    W1bT, b1b = P['W1b'].T, P['b1b'].reshape(1, D)
    W2aT, b2a = _fold_bn(P['W2a'], P['b2a'], P['g2'], P['be2'])
    W2bT, b2b = P['W2b'].T, P['b2b'].reshape(1, D)

    oh = _oh_call(bt2d)
    agg = _edge_agg(x, src, dst)
    h1, pool1 = _mlp_call(x, agg, oh, W1aT, b1a, W1bT, b1b)
    agg = _edge_agg(h1, src, dst)
    h2, pool2 = _mlp_call(h1, agg, oh, W2aT, b2a, W2bT, b2b)
    agg = _edge_agg(h2, src, dst)
    _, pool3 = _mlp_call(h2, agg, oh, W2aT, b2a, W2bT, b2b)

    hcat = jnp.concatenate([pool1, pool2, pool3], axis=1)

    WvT = P['Win'][2 * D_MODEL:].T
    bv = P['bin'][2 * D_MODEL:].reshape(1, D_MODEL)
    WoT = P['Wout'].T
    bo = P['bout'].reshape(1, D_MODEL)
    Wf1T = P['Wff1'].T
    bf1 = P['bff1'].reshape(1, -1)
    Wf2T = P['Wff2'].T
    bf2 = P['bff2'].reshape(1, D_MODEL)
    Wl1T = P['Wl1'].T
    bl1 = P['bl1'].reshape(1, D_MODEL)
    Wl2T = jnp.zeros((D_MODEL, D), jnp.float32).at[:, 0].set(P['Wl2'][0])
    bl2 = jnp.zeros((1, D), jnp.float32).at[0, 0].set(P['bl2'][0])

    out = _head_call(hcat, WvT, bv, WoT, bo,
                     P['ln1g'].reshape(1, -1), P['ln1b'].reshape(1, -1),
                     Wf1T, bf1, Wf2T, bf2,
                     P['ln2g'].reshape(1, -1), P['ln2b'].reshape(1, -1),
                     Wl1T, bl1, Wl2T, bl2)
    return out[:, :1]


# speedup vs baseline: 11.1546x; 1.1042x over previous
import functools

import jax
import jax.numpy as jnp
from jax import lax
from jax.experimental import pallas as pl
from jax.experimental.pallas import tpu as pltpu
from jax.experimental.pallas import tpu_sc as plsc

N_NODES = 10000
N_EDGES = 320000
D = 128
N_GRAPHS = 64
D_MODEL = 3 * D

NC = 2
NS = 16
NW = NC * NS
CH = 128
CPT = 78
TAIL_BASE = NW * CPT * CH
N_TAIL = (N_EDGES - TAIL_BASE) // CH
WB_ROWS = 624
ZR = 16


def _sc_agg_body(h_hbm, src_hbm, dst_hbm, out_hbm,
                 accum_sh, src_v0, src_v1, dst_v0, dst_v1,
                 dstS_v0, dstS_v1, rows_v0, rows_v1, zero_v,
                 ssem0, ssem1, dsem0, dsem1, gsem0, gsem1, csem0, csem1):
    c = lax.axis_index("c")
    s = lax.axis_index("s")
    w = c * NS + s
    ebase = w * (CPT * CH)

    src_v = (src_v0, src_v1)
    dst_v = (dst_v0, dst_v1)
    dstS_v = (dstS_v0, dstS_v1)
    rows_v = (rows_v0, rows_v1)
    ssem = (ssem0, ssem1)
    dsem = (dsem0, dsem1)
    gsem = (gsem0, gsem1)
    csem = (csem0, csem1)

    def _eoff(k):
        return pl.multiple_of(ebase + k * CH, CH)

    def _fire_idx(k, j):
        pltpu.async_copy(src_hbm.at[pl.ds(_eoff(k), CH)], src_v[j], ssem[j])
        pltpu.async_copy(dst_hbm.at[pl.ds(_eoff(k), CH)], dst_v[j], dsem[j])

    def _wait(buf, sem_):
        pltpu.make_async_copy(src_hbm.at[pl.ds(0, CH)], buf, sem_).wait()

    def _wait_rows(j):
        pltpu.make_async_copy(h_hbm.at[pl.ds(0, CH)], rows_v[j], gsem[j]).wait()

    def _fire_scatter(j):
        for m in range(CH // 16):
            dstS_v[j][pl.ds(m * 16, 16)] = dst_v[j][pl.ds(m * 16, 16)]
        pltpu.async_copy(rows_v[j], accum_sh.at[dstS_v[j]], csem[j], add=True)

    def _wait_scatter(j):
        pltpu.make_async_copy(rows_v[j], accum_sh.at[dstS_v[j]], csem[j]).wait()

    _fire_idx(0, 0)
    _fire_idx(1, 1)

    def zb(i, carry):
        for j in range(D // 16):
            zero_v[i, pl.ds(j * 16, 16)] = jnp.zeros((16,), jnp.float32)
        return carry
    lax.fori_loop(0, ZR, zb, 0)
    rbase = s * WB_ROWS
    for j in range(WB_ROWS // ZR):
        pltpu.sync_copy(zero_v, accum_sh.at[pl.ds(rbase + j * ZR, ZR)])

    @pl.when(s == NS - 1)
    def _zero_extra():
        pltpu.sync_copy(zero_v, accum_sh.at[pl.ds(NS * WB_ROWS, ZR)])

    _wait(src_v[0], ssem[0])
    pltpu.async_copy(h_hbm.at[src_v0], rows_v0, gsem[0])
    plsc.subcore_barrier()

    def _step(k, j, i=None, *, wait_prev_scatter=True, fire_gather=True,
              fire_idx=True, idx_guard=False):
        j1 = 1 - j
        if fire_gather:
            _wait(src_v[j1], ssem[j1])
            if wait_prev_scatter:
                _wait_scatter(j1)
            pltpu.async_copy(h_hbm.at[src_v[j1]], rows_v[j1], gsem[j1])
        _wait_rows(j)
        _wait(dst_v[j], dsem[j])
        _fire_scatter(j)
        if fire_idx:
            if idx_guard:
                @pl.when(i < CPT // 2 - 2)
                def _():
                    _fire_idx(k + 2, j)
            else:
                _fire_idx(k + 2, j)

    _step(0, 0, wait_prev_scatter=False)

    def body(i, carry):
        _step(2 * i + 1, 1)
        _step(2 * i + 2, 0, i, idx_guard=True)
        return carry
    lax.fori_loop(0, CPT // 2 - 1, body, 0)
    _step(CPT - 1, 1, fire_gather=False, fire_idx=False)
    _wait_scatter(0)
    _wait_scatter(1)

    @pl.when(jnp.logical_and(c == 0, s < N_TAIL))
    def _tail():
        toff = pl.multiple_of(TAIL_BASE + s * CH, CH)
        pltpu.sync_copy(src_hbm.at[pl.ds(toff, CH)], src_v0)
        pltpu.sync_copy(dst_hbm.at[pl.ds(toff, CH)], dst_v0)
        pltpu.async_copy(h_hbm.at[src_v0], rows_v0, gsem[0]).wait()
        pltpu.sync_copy(rows_v0, accum_sh.at[dst_v0], add=True)

    plsc.subcore_barrier()

    pltpu.sync_copy(accum_sh.at[pl.ds(rbase, WB_ROWS)],
                    out_hbm.at[c, pl.ds(rbase, WB_ROWS)])

    @pl.when(s == NS - 1)
    def _wb_extra():
        pltpu.sync_copy(accum_sh.at[pl.ds(NS * WB_ROWS, ZR)],
                        out_hbm.at[c, pl.ds(NS * WB_ROWS, ZR)])


@functools.cache
def _make_edge_agg():
    return functools.partial(
        pl.kernel,
        out_type=jax.ShapeDtypeStruct((NC, N_NODES, D), jnp.float32),
        mesh=plsc.VectorSubcoreMesh(core_axis_name="c", subcore_axis_name="s",
                                    num_cores=NC, num_subcores=NS),
        scratch_types=[
            pltpu.VMEM_SHARED((N_NODES, D), jnp.float32),
            pltpu.VMEM((CH,), jnp.int32),
            pltpu.VMEM((CH,), jnp.int32),
            pltpu.VMEM((CH,), jnp.int32),
            pltpu.VMEM((CH,), jnp.int32),
            pltpu.VMEM((CH,), jnp.int32),
            pltpu.VMEM((CH,), jnp.int32),
            pltpu.VMEM((CH, D), jnp.float32),
            pltpu.VMEM((CH, D), jnp.float32),
            pltpu.VMEM((ZR, D), jnp.float32),
            pltpu.SemaphoreType.DMA,
            pltpu.SemaphoreType.DMA,
            pltpu.SemaphoreType.DMA,
            pltpu.SemaphoreType.DMA,
            pltpu.SemaphoreType.DMA,
            pltpu.SemaphoreType.DMA,
            pltpu.SemaphoreType.DMA,
            pltpu.SemaphoreType.DMA,
        ],
    )(_sc_agg_body)


def _edge_agg(h, src, dst):
    return _make_edge_agg()(h, src, dst)


BLK = 2000
NBLK = N_NODES // BLK


def _oh_block(bt_ref, oh_ref):
    oh = (bt_ref[...] == lax.broadcasted_iota(jnp.int32, (BLK, N_GRAPHS), 1))
    oh_ref[...] = oh.astype(jnp.float32)


_oh_call = pl.pallas_call(
    _oh_block,
    grid=(NBLK,),
    in_specs=[pl.BlockSpec((BLK, 1), lambda i: (i, 0))],
    out_specs=pl.BlockSpec((BLK, N_GRAPHS), lambda i: (i, 0)),
    out_shape=jax.ShapeDtypeStruct((N_NODES, N_GRAPHS), jnp.float32),
)


def _mlp_block(h_ref, p_ref, oh_ref, w1_ref, b1_ref, w2_ref, b2_ref,
               hout_ref, pool_ref):
    i = pl.program_id(0)
    x = h_ref[...] + p_ref[0] + p_ref[1]
    y = lax.dot_general(x, w1_ref[...], (((1,), (0,)), ((), ())),
                        preferred_element_type=jnp.float32)
    y = jnp.maximum(y + b1_ref[...], 0.0)
    z = lax.dot_general(y, w2_ref[...], (((1,), (0,)), ((), ())),
                        preferred_element_type=jnp.float32)
    z = jnp.maximum(z + b2_ref[...], 0.0)
    hout_ref[...] = z
    pp = lax.dot_general(oh_ref[...], z, (((0,), (0,)), ((), ())),
                         preferred_element_type=jnp.float32)

    @pl.when(i == 0)
    def _():
        pool_ref[...] = pp

    @pl.when(i > 0)
    def _():
        pool_ref[...] = pool_ref[...] + pp


_mlp_call = pl.pallas_call(
    _mlp_block,
    grid=(NBLK,),
    in_specs=[
        pl.BlockSpec((BLK, D), lambda i: (i, 0)),
        pl.BlockSpec((NC, BLK, D), lambda i: (0, i, 0)),
        pl.BlockSpec((BLK, N_GRAPHS), lambda i: (i, 0)),
        pl.BlockSpec((D, D), lambda i: (0, 0)),
        pl.BlockSpec((1, D), lambda i: (0, 0)),
        pl.BlockSpec((D, D), lambda i: (0, 0)),
        pl.BlockSpec((1, D), lambda i: (0, 0)),
    ],
    out_specs=[
        pl.BlockSpec((BLK, D), lambda i: (i, 0)),
        pl.BlockSpec((N_GRAPHS, D), lambda i: (0, 0)),
    ],
    out_shape=[
        jax.ShapeDtypeStruct((N_NODES, D), jnp.float32),
        jax.ShapeDtypeStruct((N_GRAPHS, D), jnp.float32),
    ],
)


def _ln_rows(x, g, b):
    m = jnp.mean(x, axis=1, keepdims=True)
    d = x - m
    v = jnp.mean(d * d, axis=1, keepdims=True)
    return g * (d * lax.rsqrt(v + 1e-5)) + b


def _head_block(hcat_ref, wv_ref, bv_ref, wo_ref, bo_ref,
                g1_ref, be1_ref, wf1_ref, bf1_ref, wf2_ref, bf2_ref,
                g2_ref, be2_ref, wl1_ref, bl1_ref, wl2_ref, bl2_ref,
                out_ref):
    hcat = hcat_ref[...]
    v = lax.dot_general(hcat, wv_ref[...], (((1,), (0,)), ((), ())),
                        preferred_element_type=jnp.float32)
    v = v + bv_ref[...]
    a = lax.dot_general(v, wo_ref[...], (((1,), (0,)), ((), ())),
                        preferred_element_type=jnp.float32)
    a = a + bo_ref[...]
    h = _ln_rows(hcat + a, g1_ref[...], be1_ref[...])
    f = lax.dot_general(h, wf1_ref[...], (((1,), (0,)), ((), ())),
                        preferred_element_type=jnp.float32)
    f = jnp.maximum(f + bf1_ref[...], 0.0)
    f = lax.dot_general(f, wf2_ref[...], (((1,), (0,)), ((), ())),
                        preferred_element_type=jnp.float32)
    f = f + bf2_ref[...]
    h = _ln_rows(h + f, g2_ref[...], be2_ref[...])
    l = lax.dot_general(h, wl1_ref[...], (((1,), (0,)), ((), ())),
                        preferred_element_type=jnp.float32)
    l = jnp.maximum(l + bl1_ref[...], 0.0)
    o = lax.dot_general(l, wl2_ref[...], (((1,), (0,)), ((), ())),
                        preferred_element_type=jnp.float32)
    out_ref[...] = o + bl2_ref[...]


_head_call = pl.pallas_call(
    _head_block,
    out_shape=jax.ShapeDtypeStruct((N_GRAPHS, D), jnp.float32),
)


def _fold_bn(Wa, ba, g, be):
    scale = g / jnp.sqrt(1.0 + 1e-5)
    WT = (Wa * scale[:, None]).T
    b = ba * scale + be
    return WT, b.reshape(1, D)


def kernel(x, edge_index, batch, params):
    P = params
    src = jnp.asarray(edge_index[0], jnp.int32)
    dst = jnp.asarray(edge_index[1], jnp.int32)
    bt2d = jnp.asarray(batch, jnp.int32).reshape(N_NODES, 1)

    W1aT, b1a = _fold_bn(P['W1a'], P['b1a'], P['g1'], P['be1'])
    W1bT, b1b = P['W1b'].T, P['b1b'].reshape(1, D)
    W2aT, b2a = _fold_bn(P['W2a'], P['b2a'], P['g2'], P['be2'])
    W2bT, b2b = P['W2b'].T, P['b2b'].reshape(1, D)

    oh = _oh_call(bt2d)
    agg = _edge_agg(x, src, dst)
    h1, pool1 = _mlp_call(x, agg, oh, W1aT, b1a, W1bT, b1b)
    agg = _edge_agg(h1, src, dst)
    h2, pool2 = _mlp_call(h1, agg, oh, W2aT, b2a, W2bT, b2b)
    agg = _edge_agg(h2, src, dst)
    _, pool3 = _mlp_call(h2, agg, oh, W2aT, b2a, W2bT, b2b)

    hcat = jnp.concatenate([pool1, pool2, pool3], axis=1)

    WvT = P['Win'][2 * D_MODEL:].T
    bv = P['bin'][2 * D_MODEL:].reshape(1, D_MODEL)
    WoT = P['Wout'].T
    bo = P['bout'].reshape(1, D_MODEL)
    Wf1T = P['Wff1'].T
    bf1 = P['bff1'].reshape(1, -1)
    Wf2T = P['Wff2'].T
    bf2 = P['bff2'].reshape(1, D_MODEL)
    Wl1T = P['Wl1'].T
    bl1 = P['bl1'].reshape(1, D_MODEL)
    Wl2T = jnp.zeros((D_MODEL, D), jnp.float32).at[:, 0].set(P['Wl2'][0])
    bl2 = jnp.zeros((1, D), jnp.float32).at[0, 0].set(P['bl2'][0])

    out = _head_call(hcat, WvT, bv, WoT, bo,
                     P['ln1g'].reshape(1, -1), P['ln1b'].reshape(1, -1),
                     Wf1T, bf1, Wf2T, bf2,
                     P['ln2g'].reshape(1, -1), P['ln2b'].reshape(1, -1),
                     Wl1T, bl1, Wl2T, bl2)
    return out[:, :1]
